# baseline (speedup 1.0000x reference)
import functools
import jax
import jax.numpy as jnp
from jax import lax
from jax.experimental import pallas as pl
from jax.experimental.pallas import tpu as pltpu
from jax.experimental.pallas import tpu_sc as plsc

_CHUNK = 128


def _mm_body(h_ref, ws_ref, wn_ref, b_ref, hs_ref, hn_ref):
    h = h_ref[...]
    hs_ref[...] = jnp.dot(h, ws_ref[...], preferred_element_type=jnp.float32) + b_ref[...]
    hn_ref[...] = jnp.dot(h, wn_ref[...], preferred_element_type=jnp.float32) * 0.25


def _mm_call(h, ws, wn, b, T):
    N, F = h.shape
    FO = ws.shape[1]
    nn = N // T
    return pl.pallas_call(
        _mm_body,
        grid=(T,),
        in_specs=[
            pl.BlockSpec((nn, F), lambda t: (t, 0)),
            pl.BlockSpec((F, FO), lambda t: (0, 0)),
            pl.BlockSpec((F, FO), lambda t: (0, 0)),
            pl.BlockSpec((1, FO), lambda t: (0, 0)),
        ],
        out_specs=[
            pl.BlockSpec((nn, FO), lambda t: (t, 0)),
            pl.BlockSpec((nn, FO), lambda t: (t, 0)),
        ],
        out_shape=[
            jax.ShapeDtypeStruct((N, FO), jnp.float32),
            jax.ShapeDtypeStruct((N, FO), jnp.float32),
        ],
    )(h, ws, wn, b.reshape(1, FO))


def _make_sc_agg(N, F, num_cores, num_subcores):
    nw = num_cores * num_subcores
    n_per_w = N // nw
    steps = n_per_w // _CHUNK
    mesh = plsc.VectorSubcoreMesh(core_axis_name="c", subcore_axis_name="s")

    @functools.partial(
        pl.kernel,
        out_type=jax.ShapeDtypeStruct((N, F), jnp.float32),
        mesh=mesh,
        scratch_types=[
            pltpu.VMEM((4, _CHUNK), jnp.int32),
            pltpu.VMEM((4, _CHUNK, F), jnp.float32),
            pltpu.VMEM((_CHUNK, F), jnp.float32),
            pltpu.SemaphoreType.DMA,
        ],
    )
    def sc_agg(srcs_hbm, hn_hbm, hs_hbm, out_hbm, idx_v, g_v, acc_v, sem):
        wid = lax.axis_index("s") * num_cores + lax.axis_index("c")
        base0 = wid * n_per_w

        def step(s, carry):
            base = base0 + s * _CHUNK
            pltpu.sync_copy(hs_hbm.at[pl.ds(base, _CHUNK)], acc_v)
            for k in range(4):
                pltpu.sync_copy(srcs_hbm.at[pl.ds(k * N + base, _CHUNK)],
                                idx_v.at[k])
            cps = [pltpu.async_copy(hn_hbm.at[idx_v.at[k]], g_v.at[k], sem)
                   for k in range(4)]
            for cp in cps:
                cp.wait()

            def row(i, c2):
                for j in range(F // 16):
                    sl = pl.ds(j * 16, 16)
                    v = (acc_v[i, sl] + g_v[0, i, sl] + g_v[1, i, sl]
                         + g_v[2, i, sl] + g_v[3, i, sl])
                    acc_v[i, sl] = jnp.maximum(v, 0.0)
                return c2

            lax.fori_loop(0, _CHUNK, row, 0)
            pltpu.sync_copy(acc_v, out_hbm.at[pl.ds(base, _CHUNK)])
            return carry

        lax.fori_loop(0, steps, step, 0)

    return sc_agg


def kernel(x, edge_index, W1_self, W1_neigh, b1, W2_self, W2_neigh, b2):
    Bsz, T, nx, ny, F = x.shape
    FH = W1_self.shape[1]
    nn = nx * ny
    N = T * nn
    xf = x.reshape(N, F)
    srcs = edge_index[0].reshape(T, 4, nn).transpose(1, 0, 2).reshape(4 * N)

    info = plsc.get_sparse_core_info()
    sc_agg = _make_sc_agg(N, FH, info.num_cores, info.num_subcores)

    hs1, hn1 = _mm_call(xf, W1_self, W1_neigh, b1, T)
    h1 = sc_agg(srcs, hn1, hs1)
    hs2, hn2 = _mm_call(h1, W2_self, W2_neigh, b2, T)
    out = sc_agg(srcs, hn2, hs2)
    return out.reshape(Bsz, T, nx, ny, -1)


# speedup vs baseline: 1.0040x; 1.0040x over previous
import functools
import jax
import jax.numpy as jnp
from jax import lax
from jax.experimental import pallas as pl
from jax.experimental.pallas import tpu as pltpu
from jax.experimental.pallas import tpu_sc as plsc

_CHUNK = 128


def _mm_n_body(h_ref, wn_ref, hn_ref):
    hn_ref[...] = jnp.dot(h_ref[...], wn_ref[...],
                          preferred_element_type=jnp.float32) * 0.25


def _mm_s_body(h_ref, ws_ref, b_ref, hs_ref):
    hs_ref[...] = jnp.dot(h_ref[...], ws_ref[...],
                          preferred_element_type=jnp.float32) + b_ref[...]


def _combine_mm_body(hs_ref, agg_ref, wn_ref, h1_ref, hn_ref):
    h1 = jax.nn.relu(hs_ref[...] + agg_ref[...])
    h1_ref[...] = h1
    hn_ref[...] = jnp.dot(h1, wn_ref[...],
                          preferred_element_type=jnp.float32) * 0.25


def _combine_body(hs_ref, agg_ref, out_ref):
    out_ref[...] = jax.nn.relu(hs_ref[...] + agg_ref[...])


def _nf_spec(nn):
    return pl.BlockSpec((nn, None), lambda t: (t, 0))


def _w_spec():
    return pl.BlockSpec(lambda t: (0, 0))


def _tc(body, T, nn, n_in_nf, n_w, n_out, N, FO, extra_b=False, split=4):
    T, nn = T * split, nn // split
    in_specs = ([pl.BlockSpec((nn, FO), lambda t: (t, 0))] * n_in_nf
                + [pl.BlockSpec((FO, FO), lambda t: (0, 0))] * n_w
                + ([pl.BlockSpec((1, FO), lambda t: (0, 0))] if extra_b else []))
    out_specs = [pl.BlockSpec((nn, FO), lambda t: (t, 0))] * n_out
    out_shape = [jax.ShapeDtypeStruct((N, FO), jnp.float32)] * n_out
    if n_out == 1:
        out_specs, out_shape = out_specs[0], out_shape[0]
    return pl.pallas_call(body, grid=(T,), in_specs=in_specs,
                          out_specs=out_specs, out_shape=out_shape)


def _make_sc_agg(N, F, num_cores, num_subcores):
    nw = num_cores * num_subcores
    n_per_w = N // nw
    steps = n_per_w // _CHUNK
    mesh = plsc.VectorSubcoreMesh(core_axis_name="c", subcore_axis_name="s")

    @functools.partial(
        pl.kernel,
        out_type=jax.ShapeDtypeStruct((N, F), jnp.float32),
        mesh=mesh,
        scratch_types=[
            pltpu.VMEM((4, _CHUNK), jnp.int32),
            pltpu.VMEM((4, _CHUNK, F), jnp.float32),
            pltpu.SemaphoreType.DMA,
        ],
    )
    def sc_agg(srcs_hbm, hn_hbm, out_hbm, idx_v, g_v, sem):
        wid = lax.axis_index("s") * num_cores + lax.axis_index("c")
        base0 = wid * n_per_w

        def step(s, carry):
            base = base0 + s * _CHUNK
            for k in range(4):
                pltpu.sync_copy(srcs_hbm.at[pl.ds(k * N + base, _CHUNK)],
                                idx_v.at[k])
            cps = [pltpu.async_copy(hn_hbm.at[idx_v.at[k]], g_v.at[k], sem)
                   for k in range(4)]
            for cp in cps:
                cp.wait()

            def row(i, c2):
                for j in range(F // 16):
                    sl = pl.ds(j * 16, 16)
                    g_v[0, i, sl] = (g_v[0, i, sl] + g_v[1, i, sl]
                                     + g_v[2, i, sl] + g_v[3, i, sl])
                return c2

            lax.fori_loop(0, _CHUNK, row, 0)
            pltpu.sync_copy(g_v.at[0], out_hbm.at[pl.ds(base, _CHUNK)])
            return carry

        lax.fori_loop(0, steps, step, 0)

    return sc_agg


def kernel(x, edge_index, W1_self, W1_neigh, b1, W2_self, W2_neigh, b2):
    Bsz, T, nx, ny, F = x.shape
    FH = W1_self.shape[1]
    nn = nx * ny
    N = T * nn
    xf = x.reshape(N, F)
    srcs = edge_index[0].reshape(T, 4, nn).transpose(1, 0, 2).reshape(4 * N)

    info = plsc.get_sparse_core_info()
    sc_agg = _make_sc_agg(N, FH, info.num_cores, info.num_subcores)

    hn1 = _tc(_mm_n_body, T, nn, 1, 1, 1, N, FH)(xf, W1_neigh)
    agg1 = sc_agg(srcs, hn1)
    hs1 = _tc(_mm_s_body, T, nn, 1, 1, 1, N, FH, extra_b=True)(
        xf, W1_self, b1.reshape(1, FH))
    h1, hn2 = _tc(_combine_mm_body, T, nn, 2, 1, 2, N, FH)(hs1, agg1, W2_neigh)
    agg2 = sc_agg(srcs, hn2)
    hs2 = _tc(_mm_s_body, T, nn, 1, 1, 1, N, FH, extra_b=True)(
        h1, W2_self, b2.reshape(1, FH))
    out = _tc(_combine_body, T, nn, 2, 0, 1, N, FH)(hs2, agg2)
    return out.reshape(Bsz, T, nx, ny, -1)


# speedup vs baseline: 1.4023x; 1.3966x over previous
import functools
import jax
import jax.numpy as jnp
from jax import lax
from jax.experimental import pallas as pl
from jax.experimental.pallas import tpu as pltpu
from jax.experimental.pallas import tpu_sc as plsc

_CHUNK = 64


def _mm_n_body(h_ref, wn_ref, hn_ref):
    hn_ref[...] = jnp.dot(h_ref[...], wn_ref[...],
                          preferred_element_type=jnp.float32) * 0.25


def _mm_s_body(h_ref, ws_ref, b_ref, hs_ref):
    hs_ref[...] = jnp.dot(h_ref[...], ws_ref[...],
                          preferred_element_type=jnp.float32) + b_ref[...]


def _combine_mm_body(hs_ref, agg_ref, wn_ref, h1_ref, hn_ref):
    h1 = jax.nn.relu(hs_ref[...] + agg_ref[...])
    h1_ref[...] = h1
    hn_ref[...] = jnp.dot(h1, wn_ref[...],
                          preferred_element_type=jnp.float32) * 0.25


def _combine_body(hs_ref, agg_ref, out_ref):
    out_ref[...] = jax.nn.relu(hs_ref[...] + agg_ref[...])


def _nf_spec(nn):
    return pl.BlockSpec((nn, None), lambda t: (t, 0))


def _w_spec():
    return pl.BlockSpec(lambda t: (0, 0))


def _tc(body, T, nn, n_in_nf, n_w, n_out, N, FO, extra_b=False, split=4):
    T, nn = T * split, nn // split
    in_specs = ([pl.BlockSpec((nn, FO), lambda t: (t, 0))] * n_in_nf
                + [pl.BlockSpec((FO, FO), lambda t: (0, 0))] * n_w
                + ([pl.BlockSpec((1, FO), lambda t: (0, 0))] if extra_b else []))
    out_specs = [pl.BlockSpec((nn, FO), lambda t: (t, 0))] * n_out
    out_shape = [jax.ShapeDtypeStruct((N, FO), jnp.float32)] * n_out
    if n_out == 1:
        out_specs, out_shape = out_specs[0], out_shape[0]
    return pl.pallas_call(body, grid=(T,), in_specs=in_specs,
                          out_specs=out_specs, out_shape=out_shape)


def _make_sc_agg(N, F, num_cores, num_subcores):
    nw = num_cores * num_subcores
    n_per_w = N // nw
    steps = n_per_w // _CHUNK
    mesh = plsc.VectorSubcoreMesh(core_axis_name="c", subcore_axis_name="s")

    @functools.partial(
        pl.kernel,
        out_type=jax.ShapeDtypeStruct((N, F), jnp.float32),
        mesh=mesh,
        scratch_types=[
            pltpu.VMEM((4, n_per_w), jnp.int32),
            pltpu.VMEM((2, 4, _CHUNK, F), jnp.float32),
            pltpu.SemaphoreType.DMA,
            pltpu.SemaphoreType.DMA,
        ],
    )
    def sc_agg(srcs_hbm, hn_hbm, out_hbm, idx_v, g_v, sem0, sem1):
        wid = lax.axis_index("s") * num_cores + lax.axis_index("c")
        base0 = wid * n_per_w
        sems = (sem0, sem1)

        for k in range(4):
            pltpu.sync_copy(srcs_hbm.at[pl.ds(k * N + base0, n_per_w)],
                            idx_v.at[k])

        def fire(s, buf):
            off = s * _CHUNK
            for k in range(4):
                pltpu.async_copy(
                    hn_hbm.at[idx_v.at[k, pl.ds(off, _CHUNK)]],
                    g_v.at[buf, k], sems[buf])

        def drain(buf):
            for k in range(4):
                pltpu.make_async_copy(hn_hbm.at[pl.ds(0, _CHUNK)],
                                      g_v.at[buf, k], sems[buf]).wait()

        fire(0, 0)

        def process(s, buf):
            @pl.when(s + 1 < steps)
            def _():
                fire(s + 1, 1 - buf)

            drain(buf)

            def row(i, c2):
                for j in range(F // 16):
                    sl = pl.ds(j * 16, 16)
                    g_v[buf, 0, i, sl] = (g_v[buf, 0, i, sl]
                                          + g_v[buf, 1, i, sl]
                                          + g_v[buf, 2, i, sl]
                                          + g_v[buf, 3, i, sl])
                return c2

            lax.fori_loop(0, _CHUNK, row, 0)
            pltpu.sync_copy(g_v.at[buf, 0],
                            out_hbm.at[pl.ds(base0 + s * _CHUNK, _CHUNK)])

        def pair(p, carry):
            process(p * 2, 0)
            process(p * 2 + 1, 1)
            return carry

        lax.fori_loop(0, steps // 2, pair, 0)

    return sc_agg


def kernel(x, edge_index, W1_self, W1_neigh, b1, W2_self, W2_neigh, b2):
    Bsz, T, nx, ny, F = x.shape
    FH = W1_self.shape[1]
    nn = nx * ny
    N = T * nn
    xf = x.reshape(N, F)
    srcs = edge_index[0].reshape(T, 4, nn).transpose(1, 0, 2).reshape(4 * N)

    info = plsc.get_sparse_core_info()
    sc_agg = _make_sc_agg(N, FH, info.num_cores, info.num_subcores)

    hn1 = _tc(_mm_n_body, T, nn, 1, 1, 1, N, FH)(xf, W1_neigh)
    agg1 = sc_agg(srcs, hn1)
    hs1 = _tc(_mm_s_body, T, nn, 1, 1, 1, N, FH, extra_b=True)(
        xf, W1_self, b1.reshape(1, FH))
    h1, hn2 = _tc(_combine_mm_body, T, nn, 2, 1, 2, N, FH)(hs1, agg1, W2_neigh)
    agg2 = sc_agg(srcs, hn2)
    hs2 = _tc(_mm_s_body, T, nn, 1, 1, 1, N, FH, extra_b=True)(
        h1, W2_self, b2.reshape(1, FH))
    out = _tc(_combine_body, T, nn, 2, 0, 1, N, FH)(hs2, agg2)
    return out.reshape(Bsz, T, nx, ny, -1)


# speedup vs baseline: 1.5333x; 1.0934x over previous
import functools
import jax
import jax.numpy as jnp
from jax import lax
from jax.experimental import pallas as pl
from jax.experimental.pallas import tpu as pltpu
from jax.experimental.pallas import tpu_sc as plsc

_CHUNK = 64


def _mm_ns_body(h_ref, wn_ref, ws_ref, b_ref, hn_ref, hs_ref):
    h = h_ref[...]
    hn_ref[...] = jnp.dot(h, wn_ref[...],
                          preferred_element_type=jnp.float32) * 0.25
    hs_ref[...] = jnp.dot(h, ws_ref[...],
                          preferred_element_type=jnp.float32) + b_ref[...]


def _combine_mm_body(hs_ref, agg_ref, wn_ref, ws_ref, b_ref, hn_ref, hs2_ref):
    h1 = jax.nn.relu(hs_ref[...] + agg_ref[...])
    hn_ref[...] = jnp.dot(h1, wn_ref[...],
                          preferred_element_type=jnp.float32) * 0.25
    hs2_ref[...] = jnp.dot(h1, ws_ref[...],
                           preferred_element_type=jnp.float32) + b_ref[...]


def _combine_body(hs_ref, agg_ref, out_ref):
    out_ref[...] = jax.nn.relu(hs_ref[...] + agg_ref[...])


def _tc(body, T, nn, n_in_nf, n_w, n_out, N, FO, extra_b=False, split=4):
    T, nn = T * split, nn // split
    in_specs = ([pl.BlockSpec((nn, FO), lambda t: (t, 0))] * n_in_nf
                + [pl.BlockSpec((FO, FO), lambda t: (0, 0))] * n_w
                + ([pl.BlockSpec((1, FO), lambda t: (0, 0))] if extra_b else []))
    out_specs = [pl.BlockSpec((nn, FO), lambda t: (t, 0))] * n_out
    out_shape = [jax.ShapeDtypeStruct((N, FO), jnp.float32)] * n_out
    if n_out == 1:
        out_specs, out_shape = out_specs[0], out_shape[0]
    return pl.pallas_call(body, grid=(T,), in_specs=in_specs,
                          out_specs=out_specs, out_shape=out_shape)


def _make_sc_agg(N, F, num_cores, num_subcores):
    nw = num_cores * num_subcores
    n_per_w = N // nw
    steps = n_per_w // _CHUNK
    mesh = plsc.VectorSubcoreMesh(core_axis_name="c", subcore_axis_name="s")

    @functools.partial(
        pl.kernel,
        out_type=jax.ShapeDtypeStruct((N, F), jnp.float32),
        mesh=mesh,
        scratch_types=[
            pltpu.VMEM((4, n_per_w), jnp.int32),
            pltpu.VMEM((2, 4, _CHUNK, F), jnp.float32),
            pltpu.SemaphoreType.DMA,
            pltpu.SemaphoreType.DMA,
            pltpu.SemaphoreType.DMA,
        ],
    )
    def sc_agg(srcs_hbm, hn_hbm, out_hbm, idx_v, g_v, sem0, sem1, osem):
        wid = lax.axis_index("s") * num_cores + lax.axis_index("c")
        base0 = wid * n_per_w
        sems = (sem0, sem1)

        for k in range(4):
            pltpu.sync_copy(srcs_hbm.at[pl.ds(k * N + base0, n_per_w)],
                            idx_v.at[k])

        def fire(s, buf):
            off = s * _CHUNK
            for k in range(4):
                pltpu.async_copy(
                    hn_hbm.at[idx_v.at[k, pl.ds(off, _CHUNK)]],
                    g_v.at[buf, k], sems[buf])

        def drain_gathers(buf):
            for k in range(4):
                pltpu.make_async_copy(hn_hbm.at[pl.ds(0, _CHUNK)],
                                      g_v.at[buf, k], sems[buf]).wait()

        def drain_out(buf):
            pltpu.make_async_copy(hn_hbm.at[pl.ds(0, _CHUNK)],
                                  g_v.at[buf, 0], osem).wait()

        fire(0, 0)

        def process(s, buf):
            @pl.when(s >= 1)
            def _():
                drain_out(1 - buf)

            @pl.when(s + 1 < steps)
            def _():
                fire(s + 1, 1 - buf)

            drain_gathers(buf)

            def row(i, c2):
                for j in range(F // 16):
                    sl = pl.ds(j * 16, 16)
                    g_v[buf, 0, i, sl] = (g_v[buf, 0, i, sl]
                                          + g_v[buf, 1, i, sl]
                                          + g_v[buf, 2, i, sl]
                                          + g_v[buf, 3, i, sl])
                return c2

            lax.fori_loop(0, _CHUNK, row, 0)
            pltpu.async_copy(g_v.at[buf, 0],
                             out_hbm.at[pl.ds(base0 + s * _CHUNK, _CHUNK)],
                             osem)

        def pair(p, carry):
            process(p * 2, 0)
            process(p * 2 + 1, 1)
            return carry

        lax.fori_loop(0, steps // 2, pair, 0)
        drain_out(1)

    return sc_agg


def kernel(x, edge_index, W1_self, W1_neigh, b1, W2_self, W2_neigh, b2):
    Bsz, T, nx, ny, F = x.shape
    FH = W1_self.shape[1]
    nn = nx * ny
    N = T * nn
    xf = x.reshape(N, F)
    srcs = edge_index[0].reshape(T, 4, nn).transpose(1, 0, 2).reshape(4 * N)

    info = plsc.get_sparse_core_info()
    sc_agg = _make_sc_agg(N, FH, info.num_cores, info.num_subcores)

    hn1, hs1 = _tc(_mm_ns_body, T, nn, 1, 2, 2, N, FH, extra_b=True)(
        xf, W1_neigh, W1_self, b1.reshape(1, FH))
    agg1 = sc_agg(srcs, hn1)
    hn2, hs2 = _tc(_combine_mm_body, T, nn, 2, 2, 2, N, FH, extra_b=True)(
        hs1, agg1, W2_neigh, W2_self, b2.reshape(1, FH))
    agg2 = sc_agg(srcs, hn2)
    out = _tc(_combine_body, T, nn, 2, 0, 1, N, FH)(hs2, agg2)
    return out.reshape(Bsz, T, nx, ny, -1)
